# Initial kernel scaffold; baseline (speedup 1.0000x reference)
import functools

import jax
import jax.numpy as jnp
from jax import lax
from jax.experimental import pallas as pl
from jax.experimental.pallas import tpu as pltpu
from jax.experimental.pallas import tpu_sc as plsc

B = 16384
D = 128
EPS = 1e-6
L = 16

_info = plsc.get_sparse_core_info()
NC = _info.num_cores
NS = _info.num_subcores
NW = NC * NS
BPW = B // NW
CH = 128
NCHUNK = BPW // CH
GROUPS = CH // L
JD = D // L


def _sqrt16(a):
    bits = plsc.bitcast(a, jnp.int32)
    x = plsc.bitcast((bits >> 1) + 0x1FBD1DF5, jnp.float32)
    x = 0.5 * (x + a / x)
    x = 0.5 * (x + a / x)
    x = 0.5 * (x + a / x)
    return x


def _sc_body(sub_hbm, obj_hbm, idx_hbm, tab_hbm, out_hbm,
             idx_v, s_v, o_v, r_v, res_v, pt_v, sem):
    wid = lax.axis_index("s") * NC + lax.axis_index("c")
    base = wid * BPW
    pltpu.sync_copy(idx_hbm.at[pl.ds(base, BPW)], idx_v)

    lane = lax.iota(jnp.int32, L)

    def chunk_body(c, carry):
        coff = base + c * CH
        gat = pltpu.async_copy(tab_hbm.at[idx_v.at[pl.ds(c * CH, CH)]],
                               r_v, sem)
        pltpu.sync_copy(sub_hbm.at[pl.ds(coff, CH)], s_v)
        pltpu.sync_copy(obj_hbm.at[pl.ds(coff, CH)], o_v)
        gat.wait()

        def group_body(g, gcarry):
            for rr in range(L):
                r = g * L + rr
                acc = jnp.zeros((L,), jnp.float32)
                for j in range(JD):
                    cs = pl.ds(j * L, L)
                    dv = s_v[r, cs] + r_v[r, cs] - o_v[r, cs] + EPS
                    acc = acc + dv * dv
                pt_v[pl.ds(rr * L, L)] = acc
            tot = jnp.zeros((L,), jnp.float32)
            for j in range(L):
                tot = tot + plsc.load_gather(pt_v, [lane * L + j])
            res_v[pl.ds(g * L, L)] = _sqrt16(tot)
            return gcarry

        lax.fori_loop(0, GROUPS, group_body, 0)
        pltpu.sync_copy(res_v, out_hbm.at[pl.ds(coff, CH)])
        return carry

    lax.fori_loop(0, NCHUNK, chunk_body, 0)


_sc_call = pl.kernel(
    _sc_body,
    out_type=jax.ShapeDtypeStruct((B,), jnp.float32),
    mesh=plsc.VectorSubcoreMesh(core_axis_name="c", subcore_axis_name="s"),
    scratch_types=[
        pltpu.VMEM((BPW,), jnp.int32),
        pltpu.VMEM((CH, D), jnp.float32),
        pltpu.VMEM((CH, D), jnp.float32),
        pltpu.VMEM((CH, D), jnp.float32),
        pltpu.VMEM((CH,), jnp.float32),
        pltpu.VMEM((L * L,), jnp.float32),
        pltpu.SemaphoreType.DMA,
    ],
)


@jax.jit
def kernel(subject_embeddings, object_embeddings, relations, relation_weight):
    rel = relations.astype(jnp.int32)
    return _sc_call(subject_embeddings, object_embeddings, rel,
                    relation_weight)


# speedup vs baseline: 1.3595x; 1.3595x over previous
import functools

import jax
import jax.numpy as jnp
from jax import lax
from jax.experimental import pallas as pl
from jax.experimental.pallas import tpu as pltpu
from jax.experimental.pallas import tpu_sc as plsc

B = 16384
D = 128
EPS = 1e-6
L = 16

_info = plsc.get_sparse_core_info()
NC = _info.num_cores
NS = _info.num_subcores
NW = NC * NS
BPW = B // NW
CH = 128
NCHUNK = BPW // CH
GROUPS = CH // L
JD = D // L


def _sqrt16(a):
    bits = plsc.bitcast(a, jnp.int32)
    x = plsc.bitcast((bits >> 1) + 0x1FBD1DF5, jnp.float32)
    x = 0.5 * (x + a / x)
    x = 0.5 * (x + a / x)
    x = 0.5 * (x + a / x)
    return x


def _sc_body(sub_hbm, obj_hbm, idx_hbm, tab_hbm, out_hbm,
             idx_v, s_v, o_v, r_v, res_v, sem):
    wid = lax.axis_index("s") * NC + lax.axis_index("c")
    base = wid * BPW
    pltpu.sync_copy(idx_hbm.at[pl.ds(base, BPW)], idx_v)

    def chunk_body(c, carry):
        coff = base + c * CH
        gat = pltpu.async_copy(tab_hbm.at[idx_v.at[pl.ds(c * CH, CH)]],
                               r_v, sem)
        pltpu.sync_copy(sub_hbm.at[pl.ds(coff, CH)], s_v)
        pltpu.sync_copy(obj_hbm.at[pl.ds(coff, CH)], o_v)
        gat.wait()

        lane = lax.iota(jnp.int32, L)

        def group_body(g, gcarry):
            tot = jnp.zeros((L,), jnp.float32)
            for rr in range(L):
                r = g * L + rr
                acc = jnp.zeros((L,), jnp.float32)
                for j in range(JD):
                    cs = pl.ds(j * L, L)
                    dv = s_v[r, cs] + r_v[r, cs] - o_v[r, cs] + EPS
                    acc = acc + dv * dv
                tot = jnp.where(lane == rr, jnp.sum(acc), tot)
            res_v[pl.ds(g * L, L)] = _sqrt16(tot)
            return gcarry

        lax.fori_loop(0, GROUPS, group_body, 0)
        pltpu.sync_copy(res_v, out_hbm.at[pl.ds(coff, CH)])
        return carry

    lax.fori_loop(0, NCHUNK, chunk_body, 0)


_sc_call = pl.kernel(
    _sc_body,
    out_type=jax.ShapeDtypeStruct((B,), jnp.float32),
    mesh=plsc.VectorSubcoreMesh(core_axis_name="c", subcore_axis_name="s"),
    compiler_params=pltpu.CompilerParams(needs_layout_passes=False),
    scratch_types=[
        pltpu.VMEM((BPW,), jnp.int32),
        pltpu.VMEM((CH, D), jnp.float32),
        pltpu.VMEM((CH, D), jnp.float32),
        pltpu.VMEM((CH, D), jnp.float32),
        pltpu.VMEM((CH,), jnp.float32),
        pltpu.SemaphoreType.DMA,
    ],
)


@jax.jit
def kernel(subject_embeddings, object_embeddings, relations, relation_weight):
    rel = relations.astype(jnp.int32)
    return _sc_call(subject_embeddings, object_embeddings, rel,
                    relation_weight)


# speedup vs baseline: 1.5468x; 1.1378x over previous
import functools

import jax
import jax.numpy as jnp
from jax import lax
from jax.experimental import pallas as pl
from jax.experimental.pallas import tpu as pltpu
from jax.experimental.pallas import tpu_sc as plsc

B = 16384
D = 128
EPS = 1e-6
L = 16

_info = plsc.get_sparse_core_info()
NC = _info.num_cores
NS = _info.num_subcores
NW = NC * NS
BPW = B // NW
CH = 128
NCHUNK = BPW // CH
GROUPS = CH // L
JD = D // L


def _sqrt16(a):
    bits = plsc.bitcast(a, jnp.int32)
    x = plsc.bitcast((bits >> 1) + 0x1FBD1DF5, jnp.float32)
    x = 0.5 * (x + a / x)
    x = 0.5 * (x + a / x)
    x = 0.5 * (x + a / x)
    return x


def _sc_body(sub_hbm, obj_hbm, idx_hbm, tab_hbm, out_hbm,
             idx_v, s0, o0, r0, s1, o1, r1, res0, res1, sem, osem):
    wid = lax.axis_index("s") * NC + lax.axis_index("c")
    base = wid * BPW
    pltpu.sync_copy(idx_hbm.at[pl.ds(base, BPW)], idx_v)

    bufs = ((s0, o0, r0, res0), (s1, o1, r1, res1))
    lane = lax.iota(jnp.int32, L)

    def start(c, bufset):
        coff = base + c * CH
        s_v, o_v, r_v, _ = bufset
        return (
            pltpu.async_copy(tab_hbm.at[idx_v.at[pl.ds(c * CH, CH)]],
                             r_v, sem),
            pltpu.async_copy(sub_hbm.at[pl.ds(coff, CH)], s_v, sem),
            pltpu.async_copy(obj_hbm.at[pl.ds(coff, CH)], o_v, sem),
        )

    pending = start(0, bufs[0])
    out_pending = [None, None]
    for c in range(NCHUNK):
        s_v, o_v, r_v, res_v = bufs[c % 2]
        nxt = start(c + 1, bufs[(c + 1) % 2]) if c + 1 < NCHUNK else ()
        for h in pending:
            h.wait()
        pending = nxt
        if out_pending[c % 2] is not None:
            out_pending[c % 2].wait()

        def group_body(g, gcarry):
            tot = jnp.zeros((L,), jnp.float32)
            for rr in range(L):
                r = g * L + rr
                acc = jnp.zeros((L,), jnp.float32)
                for j in range(JD):
                    cs = pl.ds(j * L, L)
                    dv = s_v[r, cs] + r_v[r, cs] - o_v[r, cs] + EPS
                    acc = acc + dv * dv
                tot = jnp.where(lane == rr, jnp.sum(acc), tot)
            res_v[pl.ds(g * L, L)] = _sqrt16(tot)
            return gcarry

        lax.fori_loop(0, GROUPS, group_body, 0)
        out_pending[c % 2] = pltpu.async_copy(
            res_v, out_hbm.at[pl.ds(base + c * CH, CH)], osem)
    for h in out_pending:
        if h is not None:
            h.wait()


_sc_call = pl.kernel(
    _sc_body,
    out_type=jax.ShapeDtypeStruct((B,), jnp.float32),
    mesh=plsc.VectorSubcoreMesh(core_axis_name="c", subcore_axis_name="s"),
    compiler_params=pltpu.CompilerParams(needs_layout_passes=False),
    scratch_types=[
        pltpu.VMEM((BPW,), jnp.int32),
        pltpu.VMEM((CH, D), jnp.float32),
        pltpu.VMEM((CH, D), jnp.float32),
        pltpu.VMEM((CH, D), jnp.float32),
        pltpu.VMEM((CH, D), jnp.float32),
        pltpu.VMEM((CH, D), jnp.float32),
        pltpu.VMEM((CH, D), jnp.float32),
        pltpu.VMEM((CH,), jnp.float32),
        pltpu.VMEM((CH,), jnp.float32),
        pltpu.SemaphoreType.DMA,
        pltpu.SemaphoreType.DMA,
    ],
)


@jax.jit
def kernel(subject_embeddings, object_embeddings, relations, relation_weight):
    rel = relations.astype(jnp.int32)
    return _sc_call(subject_embeddings, object_embeddings, rel,
                    relation_weight)
